# Initial kernel scaffold; baseline (speedup 1.0000x reference)
import jax
import jax.numpy as jnp
from jax.experimental import pallas as pl


def kernel(X, Y, S, A):
    raise NotImplementedError("write your pallas kernel here")


# speedup vs baseline: 28.7607x; 28.7607x over previous
import functools

import jax
import jax.numpy as jnp
from jax.experimental import pallas as pl

N_TILE = 2048
K_TILE = 2048


def _residual_kernel(a_ref, x_ref, y_ref, r_ref):
    k = pl.program_id(0)
    part = jax.lax.dot(
        a_ref[...], x_ref[...], precision=jax.lax.Precision.HIGHEST
    )

    @pl.when(k == 0)
    def _():
        r_ref[...] = y_ref[...].T - part

    @pl.when(k != 0)
    def _():
        r_ref[...] = r_ref[...] - part


def _corr_kernel(a_ref, ry_ref, val_ref, idx_ref, g_ref, d_ref):
    n = pl.program_id(0)
    a = a_ref[...]
    cg = jax.lax.dot_general(
        a,
        ry_ref[...],
        (((0,), (0,)), ((), ())),
        precision=jax.lax.Precision.HIGHEST,
    )
    b = cg.shape[1] // 2
    corr = jnp.abs(cg[:, :b])
    gmat = cg[:, b:]

    m = jnp.max(corr, axis=0, keepdims=True)
    rows = jax.lax.broadcasted_iota(jnp.int32, corr.shape, 0)
    big = jnp.int32(corr.shape[0])
    loc = jnp.min(
        jnp.where(corr == m, rows, big), axis=0, keepdims=True
    )
    sel = rows == loc
    gsel = jnp.sum(jnp.where(sel, gmat, 0.0), axis=0, keepdims=True)
    dcol = jnp.sum(a * a, axis=0, keepdims=True)
    dsel = jnp.sum(jnp.where(sel, dcol.T, 0.0), axis=0, keepdims=True)

    val_ref[...] = m[None]
    idx_ref[...] = (loc + n * a.shape[1])[None]
    g_ref[...] = gsel[None]
    d_ref[...] = dsel[None]


def _scatter_kernel(val_ref, idx_ref, g_ref, d_ref, s_ref, h_ref, so_ref):
    n = pl.program_id(0)
    v = val_ref[:, 0, :]
    m = jnp.max(v, axis=0, keepdims=True)
    tiles = jax.lax.broadcasted_iota(jnp.int32, v.shape, 0)
    big = jnp.int32(v.shape[0])
    wt = jnp.min(jnp.where(v == m, tiles, big), axis=0, keepdims=True)
    sel = tiles == wt
    j = jnp.sum(jnp.where(sel, idx_ref[:, 0, :], 0), axis=0, keepdims=True)
    g = jnp.sum(jnp.where(sel, g_ref[:, 0, :], 0.0), axis=0, keepdims=True)
    d = jnp.sum(jnp.where(sel, d_ref[:, 0, :], 0.0), axis=0, keepdims=True)
    x = g / d

    rows = jax.lax.broadcasted_iota(jnp.int32, h_ref.shape, 0) + n * h_ref.shape[0]
    hit = rows == j
    h_ref[...] = jnp.where(hit, x, 0.0)
    so_ref[...] = hit | s_ref[...]


def kernel(X, Y, S, A):
    M, N = A.shape
    B = X.shape[1]
    n_k = N // K_TILE
    n_n = N // N_TILE

    R = pl.pallas_call(
        _residual_kernel,
        grid=(n_k,),
        in_specs=[
            pl.BlockSpec((M, K_TILE), lambda k: (0, k)),
            pl.BlockSpec((K_TILE, B), lambda k: (k, 0)),
            pl.BlockSpec((B, M), lambda k: (0, 0)),
        ],
        out_specs=pl.BlockSpec((M, B), lambda k: (0, 0)),
        out_shape=jax.ShapeDtypeStruct((M, B), A.dtype),
    )(A, X, Y)

    RY = jnp.concatenate([R, Y.T], axis=1)

    stat_shape = jax.ShapeDtypeStruct((n_n, 1, B), jnp.float32)
    stat_spec = pl.BlockSpec((1, 1, B), lambda n: (n, 0, 0))
    vals, idxs, gs, ds = pl.pallas_call(
        _corr_kernel,
        grid=(n_n,),
        in_specs=[
            pl.BlockSpec((M, N_TILE), lambda n: (0, n)),
            pl.BlockSpec((M, 2 * B), lambda n: (0, 0)),
        ],
        out_specs=[stat_spec, stat_spec, stat_spec, stat_spec],
        out_shape=[
            stat_shape,
            jax.ShapeDtypeStruct((n_n, 1, B), jnp.int32),
            stat_shape,
            stat_shape,
        ],
    )(A, RY)

    full_spec = pl.BlockSpec((n_n, 1, B), lambda n: (0, 0, 0))
    H, S_out = pl.pallas_call(
        _scatter_kernel,
        grid=(n_n,),
        in_specs=[
            full_spec,
            full_spec,
            full_spec,
            full_spec,
            pl.BlockSpec((N_TILE, B), lambda n: (n, 0)),
        ],
        out_specs=[
            pl.BlockSpec((N_TILE, B), lambda n: (n, 0)),
            pl.BlockSpec((N_TILE, B), lambda n: (n, 0)),
        ],
        out_shape=[
            jax.ShapeDtypeStruct((N, B), A.dtype),
            jax.ShapeDtypeStruct((N, B), jnp.bool_),
        ],
    )(vals, idxs, gs, ds, S)

    return (H, S_out)


# speedup vs baseline: 53.3426x; 1.8547x over previous
import functools

import jax
import jax.numpy as jnp
from jax.experimental import pallas as pl

N_TILE = 2048
K_TILE = 2048


def _residual_kernel(a_ref, x_ref, y_ref, r_ref):
    k = pl.program_id(0)
    part = jax.lax.dot(a_ref[...], x_ref[...])

    @pl.when(k == 0)
    def _():
        r_ref[...] = y_ref[...].T - part

    @pl.when(k != 0)
    def _():
        r_ref[...] = r_ref[...] - part


def _corr_kernel(a_ref, ry_ref, val_ref, idx_ref, g_ref, d_ref):
    n = pl.program_id(0)
    a = a_ref[...]
    cg = jax.lax.dot_general(a, ry_ref[...], (((0,), (0,)), ((), ())))
    b = cg.shape[1] // 2
    corr = jnp.abs(cg[:, :b])
    gmat = cg[:, b:]

    m = jnp.max(corr, axis=0, keepdims=True)
    rows = jax.lax.broadcasted_iota(jnp.int32, corr.shape, 0)
    big = jnp.int32(corr.shape[0])
    loc = jnp.min(
        jnp.where(corr == m, rows, big), axis=0, keepdims=True
    )
    sel = rows == loc
    gsel = jnp.sum(jnp.where(sel, gmat, 0.0), axis=0, keepdims=True)
    dcol = jnp.sum(a * a, axis=0, keepdims=True)
    dsel = jnp.sum(jnp.where(sel, dcol.T, 0.0), axis=0, keepdims=True)

    val_ref[...] = m[None]
    idx_ref[...] = (loc + n * a.shape[1])[None]
    g_ref[...] = gsel[None]
    d_ref[...] = dsel[None]


def _scatter_kernel(val_ref, idx_ref, g_ref, d_ref, s_ref, h_ref, so_ref):
    n = pl.program_id(0)
    v = val_ref[:, 0, :]
    m = jnp.max(v, axis=0, keepdims=True)
    tiles = jax.lax.broadcasted_iota(jnp.int32, v.shape, 0)
    big = jnp.int32(v.shape[0])
    wt = jnp.min(jnp.where(v == m, tiles, big), axis=0, keepdims=True)
    sel = tiles == wt
    j = jnp.sum(jnp.where(sel, idx_ref[:, 0, :], 0), axis=0, keepdims=True)
    g = jnp.sum(jnp.where(sel, g_ref[:, 0, :], 0.0), axis=0, keepdims=True)
    d = jnp.sum(jnp.where(sel, d_ref[:, 0, :], 0.0), axis=0, keepdims=True)
    x = g / d

    rows = jax.lax.broadcasted_iota(jnp.int32, h_ref.shape, 0) + n * h_ref.shape[0]
    hit = rows == j
    h_ref[...] = jnp.where(hit, x, 0.0)
    so_ref[...] = hit | s_ref[...]


def kernel(X, Y, S, A):
    M, N = A.shape
    B = X.shape[1]
    n_k = N // K_TILE
    n_n = N // N_TILE

    R = pl.pallas_call(
        _residual_kernel,
        grid=(n_k,),
        in_specs=[
            pl.BlockSpec((M, K_TILE), lambda k: (0, k)),
            pl.BlockSpec((K_TILE, B), lambda k: (k, 0)),
            pl.BlockSpec((B, M), lambda k: (0, 0)),
        ],
        out_specs=pl.BlockSpec((M, B), lambda k: (0, 0)),
        out_shape=jax.ShapeDtypeStruct((M, B), A.dtype),
    )(A, X, Y)

    RY = jnp.concatenate([R, Y.T], axis=1)

    stat_shape = jax.ShapeDtypeStruct((n_n, 1, B), jnp.float32)
    stat_spec = pl.BlockSpec((1, 1, B), lambda n: (n, 0, 0))
    vals, idxs, gs, ds = pl.pallas_call(
        _corr_kernel,
        grid=(n_n,),
        in_specs=[
            pl.BlockSpec((M, N_TILE), lambda n: (0, n)),
            pl.BlockSpec((M, 2 * B), lambda n: (0, 0)),
        ],
        out_specs=[stat_spec, stat_spec, stat_spec, stat_spec],
        out_shape=[
            stat_shape,
            jax.ShapeDtypeStruct((n_n, 1, B), jnp.int32),
            stat_shape,
            stat_shape,
        ],
    )(A, RY)

    full_spec = pl.BlockSpec((n_n, 1, B), lambda n: (0, 0, 0))
    H, S_out = pl.pallas_call(
        _scatter_kernel,
        grid=(n_n,),
        in_specs=[
            full_spec,
            full_spec,
            full_spec,
            full_spec,
            pl.BlockSpec((N_TILE, B), lambda n: (n, 0)),
        ],
        out_specs=[
            pl.BlockSpec((N_TILE, B), lambda n: (n, 0)),
            pl.BlockSpec((N_TILE, B), lambda n: (n, 0)),
        ],
        out_shape=[
            jax.ShapeDtypeStruct((N, B), A.dtype),
            jax.ShapeDtypeStruct((N, B), jnp.bool_),
        ],
    )(vals, idxs, gs, ds, S)

    return (H, S_out)
